# baseline (speedup 1.0000x reference)
import functools

import jax
import jax.numpy as jnp
from jax import lax
from jax.experimental import pallas as pl
from jax.experimental.pallas import tpu as pltpu
from jax.experimental.pallas import tpu_sc as plsc

BSZ = 64
NPG = 64
EPG = 1024
IN_FEAT = 128
HID = 128
EF = 16
HEADS = 8
POS = NPG * NPG
F32 = jnp.float32

NUM_TILES = 32
GPT = BSZ // NUM_TILES
OPT = GPT * POS
EPT = GPT * EPG
LANES = 16


BF16 = jnp.bfloat16


def _dot(a, b):
    return jnp.dot(a.astype(BF16), b.astype(BF16), preferred_element_type=F32)


def _dotx(a, b):
    return jnp.dot(a, b, preferred_element_type=F32,
                   precision=lax.Precision.HIGHEST)


def _graph_kernel(x_ref, ea_ref, srcc_ref, dstc_ref, dstr_ref, keyc_ref,
                  keyr_ref, atomW_ref, atomb_ref, bondW_ref, bondb_ref,
                  gW_ref, gb_ref, W1a_ref, W1e_ref, b1_ref, W2_ref, b2_ref,
                  outd_ref, ve_ref, idx_ref):
    h = _dot(x_ref[...], atomW_ref[...]) + atomb_ref[...]
    ea = _dot(ea_ref[...], bondW_ref[...]) + bondb_ref[...]

    srcc = srcc_ref[0]
    dstc = dstc_ref[0]
    dstr = dstr_ref[0]
    col_n = lax.broadcasted_iota(jnp.int32, (EPG, NPG), 1)
    src_oh = jnp.where(srcc == col_n, 1.0, 0.0)
    dst_oh = jnp.where(dstc == col_n, 1.0, 0.0)
    row_n = lax.broadcasted_iota(jnp.int32, (NPG, EPG), 0)
    dst_ohT = jnp.where(row_n == dstr, 1.0, 0.0)

    for l in range(3):
        t = jax.nn.relu(_dot(ea, gW_ref[4 * l]) + gb_ref[4 * l])
        e = _dot(t, gW_ref[4 * l + 1]) + gb_ref[4 * l + 1]
        msg = jax.nn.relu(_dotx(src_oh, h) + e)
        aggr = _dotx(dst_ohT, msg)
        h2 = h + aggr
        h2 = jax.nn.relu(_dot(h2, gW_ref[4 * l + 2]) + gb_ref[4 * l + 2])
        h2 = _dot(h2, gW_ref[4 * l + 3]) + gb_ref[4 * l + 3]
        if l < 2:
            h2 = jax.nn.relu(h2)
        h = h2

    W1a = W1a_ref[...]
    b1 = b1_ref[...]
    W2 = W2_ref[...]
    b2 = b2_ref[...]
    sel = jnp.where(
        lax.broadcasted_iota(jnp.int32, (HID, HEADS), 0) % HEADS
        == lax.broadcasted_iota(jnp.int32, (HID, HEADS), 1), 1.0, 0.0)

    hb = h.astype(BF16).astype(F32)
    hn = jnp.broadcast_to(hb[:, None, :], (NPG, NPG, HID)).reshape(POS, HID)
    hm = jnp.broadcast_to(hb[None, :, :], (NPG, NPG, HID)).reshape(POS, HID)
    attn8 = _dotx(hn * hm, sel)
    zd = _dot(attn8, W1a) + b1
    outd_ref[...] = _dot(jax.nn.relu(zd), W2) + b2

    hs = _dotx(src_oh, hb)
    hd = _dotx(dst_oh, hb)
    attn8e = _dotx(hs * hd, sel)
    ze = _dot(attn8e, W1a) + b1 + _dot(ea, W1e_ref[...])
    ve_ref[...] = _dot(jax.nn.relu(ze), W2) + b2

    keyc = keyc_ref[0]
    keyr = keyr_ref[0]
    e_row = lax.broadcasted_iota(jnp.int32, (EPG, EPG), 0)
    e_col = lax.broadcasted_iota(jnp.int32, (EPG, EPG), 1)
    later_dup = (keyc == keyr) & (e_col > e_row)
    dupcnt = jnp.sum(jnp.where(later_dup, 1.0, 0.0), axis=1, keepdims=True)
    off = (lax.rem(pl.program_id(0), GPT)) * POS
    idx_ref[...] = jnp.where(dupcnt == 0.0, keyc + off, OPT)


def _scatter_sc_kernel(outd_hbm, idx_hbm, ve_hbm, out_hbm,
                       dense_v, idx_v, ve_v):
    wid = lax.axis_index("s") * 2 + lax.axis_index("c")
    pltpu.sync_copy(outd_hbm.at[pl.ds(wid * OPT, OPT)], dense_v)
    pltpu.sync_copy(idx_hbm.at[pl.ds(wid * EPT, EPT)], idx_v)
    pltpu.sync_copy(ve_hbm.at[pl.ds(wid * EPT, EPT)], ve_v)
    for j in range(EPT // LANES):
        iv = idx_v[pl.ds(j * LANES, LANES)]
        vv = ve_v[pl.ds(j * LANES, LANES)]
        plsc.store_scatter(dense_v, [iv], vv, mask=iv < OPT)
    pltpu.sync_copy(dense_v, out_hbm.at[pl.ds(wid * OPT, OPT)])


def _scatter_sc(outd, idx, ve):
    run = functools.partial(
        pl.kernel,
        out_type=jax.ShapeDtypeStruct((BSZ * POS,), F32),
        mesh=plsc.VectorSubcoreMesh(core_axis_name="c", subcore_axis_name="s"),
        compiler_params=pltpu.CompilerParams(needs_layout_passes=False),
        scratch_types=[
            pltpu.VMEM((OPT,), F32),
            pltpu.VMEM((EPT,), jnp.int32),
            pltpu.VMEM((EPT,), F32),
        ],
    )(_scatter_sc_kernel)
    return run(outd, idx, ve)


def _tc_stage(x, edge_attr, params, edge_index, batch):
    src = edge_index[0]
    dst = edge_index[1]
    src_l = jnp.remainder(src, NPG).astype(jnp.int32)
    dst_l = jnp.remainder(dst, NPG).astype(jnp.int32)
    key = src_l * NPG + dst_l
    srcc = src_l.reshape(BSZ, EPG, 1)
    dstc = dst_l.reshape(BSZ, EPG, 1)
    dstr = dst_l.reshape(BSZ, 1, EPG)
    keyc = key.reshape(BSZ, EPG, 1)
    keyr = key.reshape(BSZ, 1, EPG)

    p = params
    gWs = jnp.stack([p['gnn'][l][k] for l in range(3)
                     for k in ('be_W1', 'be_W2', 'nn_W1', 'nn_W2')])
    gbs = jnp.stack([p['gnn'][l][k] for l in range(3)
                     for k in ('be_b1', 'be_b2', 'nn_b1', 'nn_b2')])
    gbs = gbs.reshape(12, 1, HID)
    fd = HID // HEADS
    W1a = p['mlp_W1'][:HEADS] * (1.0 / (fd ** 0.5))
    W1e = p['mlp_W1'][HEADS:]
    b1 = p['mlp_b1'].reshape(1, HID)
    W2 = p['mlp_W2']
    b2 = p['mlp_b2'].reshape(1, 1)
    atomb = p['atom_b'].reshape(1, HID)
    bondb = p['bond_b'].reshape(1, HID)

    full2 = lambda g: (0, 0)
    full3 = lambda g: (0, 0, 0)
    outd, ve, idx = pl.pallas_call(
        _graph_kernel,
        grid=(BSZ,),
        in_specs=[
            pl.BlockSpec((NPG, IN_FEAT), lambda g: (g, 0)),
            pl.BlockSpec((EPG, EF), lambda g: (g, 0)),
            pl.BlockSpec((1, EPG, 1), lambda g: (g, 0, 0)),
            pl.BlockSpec((1, EPG, 1), lambda g: (g, 0, 0)),
            pl.BlockSpec((1, 1, EPG), lambda g: (g, 0, 0)),
            pl.BlockSpec((1, EPG, 1), lambda g: (g, 0, 0)),
            pl.BlockSpec((1, 1, EPG), lambda g: (g, 0, 0)),
            pl.BlockSpec((IN_FEAT, HID), full2),
            pl.BlockSpec((1, HID), full2),
            pl.BlockSpec((EF, HID), full2),
            pl.BlockSpec((1, HID), full2),
            pl.BlockSpec((12, HID, HID), full3),
            pl.BlockSpec((12, 1, HID), full3),
            pl.BlockSpec((HEADS, HID), full2),
            pl.BlockSpec((HID, HID), full2),
            pl.BlockSpec((1, HID), full2),
            pl.BlockSpec((HID, 1), full2),
            pl.BlockSpec((1, 1), full2),
        ],
        out_specs=[
            pl.BlockSpec((POS, 1), lambda g: (g, 0)),
            pl.BlockSpec((EPG, 1), lambda g: (g, 0)),
            pl.BlockSpec((EPG, 1), lambda g: (g, 0)),
        ],
        out_shape=[
            jax.ShapeDtypeStruct((BSZ * POS, 1), F32),
            jax.ShapeDtypeStruct((BSZ * EPG, 1), F32),
            jax.ShapeDtypeStruct((BSZ * EPG, 1), jnp.int32),
        ],
    )(x, edge_attr, srcc, dstc, dstr, keyc, keyr,
      p['atom_W'], atomb, p['bond_W'], bondb, gWs, gbs,
      W1a, W1e, b1, W2, b2)
    return outd, ve, idx


def kernel(x, edge_attr, params, edge_index, batch):
    outd, ve, idx = _tc_stage(x, edge_attr, params, edge_index, batch)
    out = _scatter_sc(outd.reshape(BSZ * POS),
                      idx.reshape(BSZ * EPG),
                      ve.reshape(BSZ * EPG))
    emb = out.reshape(BSZ, NPG, NPG, 1)
    mask = jnp.ones((BSZ, NPG, NPG), F32)
    return emb, mask
